# baseline (speedup 1.0000x reference)
import functools

import jax
import jax.numpy as jnp
from jax import lax
from jax.experimental import pallas as pl
from jax.experimental.pallas import tpu as pltpu
from jax.experimental.pallas import tpu_sc as plsc

_N_BINS = 10


def _row_stats(yv, lab_row):
    _, nclass = yv.shape
    yt = yv.T
    m = jnp.max(yt, axis=0, keepdims=True)
    e = jnp.exp(yt - m)
    ones_row = jnp.ones((1, nclass), jnp.float32)
    s = lax.dot_general(ones_row, e, (((1,), (0,)), ((), ())),
                        preferred_element_type=jnp.float32)
    ismax = (yt == m).astype(jnp.float32)
    iota_row = lax.broadcasted_iota(jnp.int32, (1, nclass), 1).astype(jnp.float32)
    predf = lax.dot_general(iota_row, ismax, (((1,), (0,)), ((), ())),
                            preferred_element_type=jnp.float32)
    lab_f = lab_row.astype(jnp.float32)
    return 1.0 / s, (predf == lab_f).astype(jnp.float32)


def _stage1_body(y_ref, lab_ref, conf_ref, acc_ref):
    conf, acc = _row_stats(y_ref[...], lab_ref[0])
    conf_ref[0] = conf
    acc_ref[0] = acc


def _stage1(y, labels3, block_rows, block_off, nb):
    _, c = y.shape
    vec_spec = pl.BlockSpec((1, 1, block_rows), lambda i: (i, 0, 0))
    conf3, acc3 = pl.pallas_call(
        _stage1_body,
        grid=(nb,),
        in_specs=[
            pl.BlockSpec((block_rows, c), lambda i: (i + block_off, 0)),
            pl.BlockSpec((1, 1, block_rows), lambda i: (i + block_off, 0, 0)),
        ],
        out_specs=[vec_spec] * 2,
        out_shape=[jax.ShapeDtypeStruct((nb, 1, block_rows), jnp.float32)] * 2,
    )(y, labels3)
    return conf3.reshape(-1), acc3.reshape(-1)


def _make_hist(n):
    nw = 32
    per_w = n // nw
    n_vec = per_w // 16
    mesh = plsc.VectorSubcoreMesh(core_axis_name="c", subcore_axis_name="s")

    half_w = per_w // 2
    scratch = [pltpu.VMEM((half_w,), jnp.float32) for _ in range(4)]

    @functools.partial(
        pl.kernel,
        mesh=mesh,
        out_type=jax.ShapeDtypeStruct((nw, (2 + 3 * (_N_BINS - 1)) * 16),
                                      jnp.float32),
        scratch_types=scratch + [
            pltpu.VMEM((_N_BINS + 1, 16), jnp.float32),
            pltpu.VMEM(((2 + 3 * (_N_BINS - 1)) * 16,), jnp.float32),
            pltpu.SemaphoreType.DMA,
            pltpu.SemaphoreType.DMA,
        ],
    )
    def hist(conf_hbm, acc_hbm, bounds_hbm, out_hbm, conf_v0, conf_v1, acc_v0,
             acc_v1, bounds_v, accum_v, sem0, sem1):
        wid = lax.axis_index("s") * 2 + lax.axis_index("c")
        base = wid * per_w
        cp = [
            pltpu.async_copy(conf_hbm.at[pl.ds(base, half_w)], conf_v0, sem0),
            pltpu.async_copy(acc_hbm.at[pl.ds(base, half_w)], acc_v0, sem0),
            pltpu.async_copy(conf_hbm.at[pl.ds(base + half_w, half_w)],
                             conf_v1, sem1),
            pltpu.async_copy(acc_hbm.at[pl.ds(base + half_w, half_w)],
                             acc_v1, sem1),
        ]
        pltpu.sync_copy(bounds_hbm, bounds_v)
        zeros16 = jnp.zeros((16,), jnp.float32)
        ones16 = jnp.ones((16,), jnp.float32)
        thr = [bounds_v[k] for k in range(1, _N_BINS)]

        def accumulate(carry, v, a):
            out = [carry[0] + v, carry[1] + a]
            for k in range(_N_BINS - 1):
                m = v > thr[k]
                out.append(carry[3 * k + 2] + jnp.where(m, ones16, zeros16))
                out.append(carry[3 * k + 3] + jnp.where(m, v, zeros16))
                out.append(carry[3 * k + 4] + jnp.where(m, a, zeros16))
            return tuple(out)

        def make_body(conf_v, acc_v):
            def body(i, carry):
                v0 = conf_v[pl.ds(i * 32, 16)]
                a0 = acc_v[pl.ds(i * 32, 16)]
                v1 = conf_v[pl.ds(i * 32 + 16, 16)]
                a1 = acc_v[pl.ds(i * 32 + 16, 16)]
                return accumulate(accumulate(carry, v0, a0), v1, a1)
            return body

        init = tuple(zeros16 for _ in range(2 + 3 * (_N_BINS - 1)))
        cp[0].wait()
        cp[1].wait()
        mid = lax.fori_loop(0, half_w // 32, make_body(conf_v0, acc_v0), init)
        cp[2].wait()
        cp[3].wait()
        final = lax.fori_loop(0, half_w // 32, make_body(conf_v1, acc_v1), mid)
        for j in range(2 + 3 * (_N_BINS - 1)):
            accum_v[pl.ds(j * 16, 16)] = final[j]
        pltpu.sync_copy(accum_v, out_hbm.at[wid])

    return hist


def kernel(y, labels):
    n, _ = y.shape
    block_rows = 32768
    n_chunks = 4
    nb_total = n // block_rows
    nb = nb_total // n_chunks
    labels3 = labels.reshape(nb_total, 1, block_rows)
    bounds = jnp.linspace(0.0, 1.0, _N_BINS + 1)
    bounds_b = jnp.broadcast_to(bounds[:, None], (_N_BINS + 1, 16))
    hist = _make_hist(n // n_chunks)
    partial_list = []
    for ci in range(n_chunks):
        conf_flat, acc_flat = _stage1(y, labels3, block_rows, ci * nb, nb)
        partial_list.append(hist(conf_flat, acc_flat, bounds_b))
    partials = sum(partial_list)
    p = partials.reshape(32, 2 + 3 * (_N_BINS - 1), 16).sum(axis=(0, 2))
    tot_v, tot_a = p[0], p[1]
    exc = p[2:].reshape(_N_BINS - 1, 3)
    c_exc = jnp.concatenate([jnp.array([float(n)]), exc[:, 0],
                             jnp.array([0.0])])
    s_exc = jnp.concatenate([tot_v[None], exc[:, 1], jnp.array([0.0])])
    a_exc = jnp.concatenate([tot_a[None], exc[:, 2], jnp.array([0.0])])
    cnt = c_exc[:-1] - c_exc[1:]
    sconf = s_exc[:-1] - s_exc[1:]
    sacc = a_exc[:-1] - a_exc[1:]
    denom = jnp.maximum(cnt, 1.0)
    contrib = jnp.abs(sconf / denom - sacc / denom) * (cnt / n)
    ece = jnp.sum(jnp.where(cnt > 0, contrib, 0.0))
    return ece.reshape(1)


# speedup vs baseline: 1.0145x; 1.0145x over previous
import functools

import jax
import jax.numpy as jnp
from jax import lax
from jax.experimental import pallas as pl
from jax.experimental.pallas import tpu as pltpu
from jax.experimental.pallas import tpu_sc as plsc

_N_BINS = 10


def _row_stats(yv, lab_row):
    _, nclass = yv.shape
    yt = yv.T
    m = jnp.max(yt, axis=0, keepdims=True)
    e = jnp.exp(yt - m)
    ones_row = jnp.ones((1, nclass), jnp.float32)
    s = lax.dot_general(ones_row, e, (((1,), (0,)), ((), ())),
                        preferred_element_type=jnp.float32)
    ismax = (yt == m).astype(jnp.float32)
    iota_row = lax.broadcasted_iota(jnp.int32, (1, nclass), 1).astype(jnp.float32)
    predf = lax.dot_general(iota_row, ismax, (((1,), (0,)), ((), ())),
                            preferred_element_type=jnp.float32)
    lab_f = lab_row.astype(jnp.float32)
    return 1.0 / s, (predf == lab_f).astype(jnp.float32)


def _stage1_body(y_ref, lab_ref, conf_ref, acc_ref):
    conf, acc = _row_stats(y_ref[...], lab_ref[0])
    conf_ref[0] = conf
    acc_ref[0] = acc


def _stage1(y, labels3, block_rows, block_off, nb):
    _, c = y.shape
    vec_spec = pl.BlockSpec((1, 1, block_rows), lambda i: (i, 0, 0))
    conf3, acc3 = pl.pallas_call(
        _stage1_body,
        grid=(nb,),
        in_specs=[
            pl.BlockSpec((block_rows, c), lambda i: (i + block_off, 0)),
            pl.BlockSpec((1, 1, block_rows), lambda i: (i + block_off, 0, 0)),
        ],
        out_specs=[vec_spec] * 2,
        out_shape=[jax.ShapeDtypeStruct((nb, 1, block_rows), jnp.float32)] * 2,
    )(y, labels3)
    return conf3.reshape(-1), acc3.reshape(-1)


def _make_hist(n):
    nw = 32
    per_w = n // nw
    n_vec = per_w // 16
    mesh = plsc.VectorSubcoreMesh(core_axis_name="c", subcore_axis_name="s")

    half_w = per_w // 2
    scratch = [pltpu.VMEM((half_w,), jnp.float32) for _ in range(4)]

    @functools.partial(
        pl.kernel,
        mesh=mesh,
        out_type=jax.ShapeDtypeStruct((nw, (2 + 3 * (_N_BINS - 1)) * 16),
                                      jnp.float32),
        scratch_types=scratch + [
            pltpu.VMEM((_N_BINS + 1, 16), jnp.float32),
            pltpu.VMEM(((2 + 3 * (_N_BINS - 1)) * 16,), jnp.float32),
            pltpu.SemaphoreType.DMA,
            pltpu.SemaphoreType.DMA,
        ],
    )
    def hist(conf_hbm, acc_hbm, bounds_hbm, out_hbm, conf_v0, conf_v1, acc_v0,
             acc_v1, bounds_v, accum_v, sem0, sem1):
        wid = lax.axis_index("s") * 2 + lax.axis_index("c")
        base = wid * per_w
        cp = [
            pltpu.async_copy(conf_hbm.at[pl.ds(base, half_w)], conf_v0, sem0),
            pltpu.async_copy(acc_hbm.at[pl.ds(base, half_w)], acc_v0, sem0),
            pltpu.async_copy(conf_hbm.at[pl.ds(base + half_w, half_w)],
                             conf_v1, sem1),
            pltpu.async_copy(acc_hbm.at[pl.ds(base + half_w, half_w)],
                             acc_v1, sem1),
        ]
        pltpu.sync_copy(bounds_hbm, bounds_v)
        zeros16 = jnp.zeros((16,), jnp.float32)
        ones16 = jnp.ones((16,), jnp.float32)
        thr = [bounds_v[k] for k in range(1, _N_BINS)]

        def accumulate(carry, v, a):
            out = [carry[0] + v, carry[1] + a]
            for k in range(_N_BINS - 1):
                m = v > thr[k]
                out.append(carry[3 * k + 2] + jnp.where(m, ones16, zeros16))
                out.append(carry[3 * k + 3] + jnp.where(m, v, zeros16))
                out.append(carry[3 * k + 4] + jnp.where(m, a, zeros16))
            return tuple(out)

        def make_body(conf_v, acc_v):
            def body(i, carry):
                v0 = conf_v[pl.ds(i * 32, 16)]
                a0 = acc_v[pl.ds(i * 32, 16)]
                v1 = conf_v[pl.ds(i * 32 + 16, 16)]
                a1 = acc_v[pl.ds(i * 32 + 16, 16)]
                return accumulate(accumulate(carry, v0, a0), v1, a1)
            return body

        init = tuple(zeros16 for _ in range(2 + 3 * (_N_BINS - 1)))
        cp[0].wait()
        cp[1].wait()
        mid = lax.fori_loop(0, half_w // 32, make_body(conf_v0, acc_v0), init)
        cp[2].wait()
        cp[3].wait()
        final = lax.fori_loop(0, half_w // 32, make_body(conf_v1, acc_v1), mid)
        for j in range(2 + 3 * (_N_BINS - 1)):
            accum_v[pl.ds(j * 16, 16)] = final[j]
        pltpu.sync_copy(accum_v, out_hbm.at[wid])

    return hist


def kernel(y, labels):
    n, _ = y.shape
    block_rows = 32768
    n_chunks = 1
    nb_total = n // block_rows
    nb = nb_total // n_chunks
    labels3 = labels.reshape(nb_total, 1, block_rows)
    bounds = jnp.linspace(0.0, 1.0, _N_BINS + 1)
    bounds_b = jnp.broadcast_to(bounds[:, None], (_N_BINS + 1, 16))
    hist = _make_hist(n // n_chunks)
    partial_list = []
    for ci in range(n_chunks):
        conf_flat, acc_flat = _stage1(y, labels3, block_rows, ci * nb, nb)
        partial_list.append(hist(conf_flat, acc_flat, bounds_b))
    partials = sum(partial_list)
    p = partials.reshape(32, 2 + 3 * (_N_BINS - 1), 16).sum(axis=(0, 2))
    tot_v, tot_a = p[0], p[1]
    exc = p[2:].reshape(_N_BINS - 1, 3)
    c_exc = jnp.concatenate([jnp.array([float(n)]), exc[:, 0],
                             jnp.array([0.0])])
    s_exc = jnp.concatenate([tot_v[None], exc[:, 1], jnp.array([0.0])])
    a_exc = jnp.concatenate([tot_a[None], exc[:, 2], jnp.array([0.0])])
    cnt = c_exc[:-1] - c_exc[1:]
    sconf = s_exc[:-1] - s_exc[1:]
    sacc = a_exc[:-1] - a_exc[1:]
    denom = jnp.maximum(cnt, 1.0)
    contrib = jnp.abs(sconf / denom - sacc / denom) * (cnt / n)
    ece = jnp.sum(jnp.where(cnt > 0, contrib, 0.0))
    return ece.reshape(1)


# speedup vs baseline: 1.0379x; 1.0231x over previous
import functools

import jax
import jax.numpy as jnp
from jax import lax
from jax.experimental import pallas as pl
from jax.experimental.pallas import tpu as pltpu
from jax.experimental.pallas import tpu_sc as plsc

_N_BINS = 10


def _row_stats(yv, lab_row):
    _, nclass = yv.shape
    yt = yv.T
    m = jnp.max(yt, axis=0, keepdims=True)
    e = jnp.exp(yt - m)
    ones_row = jnp.ones((1, nclass), jnp.float32)
    s = lax.dot_general(ones_row, e, (((1,), (0,)), ((), ())),
                        preferred_element_type=jnp.float32)
    ismax = (yt == m).astype(jnp.float32)
    iota_row = lax.broadcasted_iota(jnp.int32, (1, nclass), 1).astype(jnp.float32)
    predf = lax.dot_general(iota_row, ismax, (((1,), (0,)), ((), ())),
                            preferred_element_type=jnp.float32)
    lab_f = lab_row.astype(jnp.float32)
    return 1.0 / s, (predf == lab_f).astype(jnp.float32)


def _stage1_body(y_ref, lab_ref, conf_ref, acc_ref):
    conf, acc = _row_stats(y_ref[...], lab_ref[0])
    conf_ref[0] = conf
    acc_ref[0] = acc


def _stage1(y, labels3, block_rows, block_off, nb):
    _, c = y.shape
    vec_spec = pl.BlockSpec((1, 1, block_rows), lambda i: (i, 0, 0))
    conf3, acc3 = pl.pallas_call(
        _stage1_body,
        grid=(nb,),
        in_specs=[
            pl.BlockSpec((block_rows, c), lambda i: (i + block_off, 0)),
            pl.BlockSpec((1, 1, block_rows), lambda i: (i + block_off, 0, 0)),
        ],
        out_specs=[vec_spec] * 2,
        out_shape=[jax.ShapeDtypeStruct((nb, 1, block_rows), jnp.float32)] * 2,
    )(y, labels3)
    return conf3.reshape(-1), acc3.reshape(-1)


def _make_hist(n):
    nw = 32
    per_w = n // nw
    n_vec = per_w // 16
    mesh = plsc.VectorSubcoreMesh(core_axis_name="c", subcore_axis_name="s")

    half_w = per_w // 2
    scratch = [pltpu.VMEM((half_w,), jnp.float32) for _ in range(4)]

    @functools.partial(
        pl.kernel,
        mesh=mesh,
        out_type=jax.ShapeDtypeStruct((nw, (2 + 3 * (_N_BINS - 1)) * 16),
                                      jnp.float32),
        scratch_types=scratch + [
            pltpu.VMEM((_N_BINS + 1, 16), jnp.float32),
            pltpu.VMEM(((2 + 3 * (_N_BINS - 1)) * 16,), jnp.float32),
            pltpu.SemaphoreType.DMA,
            pltpu.SemaphoreType.DMA,
        ],
    )
    def hist(conf_hbm, acc_hbm, bounds_hbm, out_hbm, conf_v0, conf_v1, acc_v0,
             acc_v1, bounds_v, accum_v, sem0, sem1):
        wid = lax.axis_index("s") * 2 + lax.axis_index("c")
        base = wid * per_w
        cp = [
            pltpu.async_copy(conf_hbm.at[pl.ds(base, half_w)], conf_v0, sem0),
            pltpu.async_copy(acc_hbm.at[pl.ds(base, half_w)], acc_v0, sem0),
            pltpu.async_copy(conf_hbm.at[pl.ds(base + half_w, half_w)],
                             conf_v1, sem1),
            pltpu.async_copy(acc_hbm.at[pl.ds(base + half_w, half_w)],
                             acc_v1, sem1),
        ]
        pltpu.sync_copy(bounds_hbm, bounds_v)
        zeros16 = jnp.zeros((16,), jnp.float32)
        ones16 = jnp.ones((16,), jnp.float32)
        thr = [bounds_v[k] for k in range(1, _N_BINS)]

        def accumulate(carry, v, a):
            out = [carry[0] + v, carry[1] + a]
            for k in range(_N_BINS - 1):
                m = v > thr[k]
                out.append(carry[3 * k + 2] + jnp.where(m, ones16, zeros16))
                out.append(carry[3 * k + 3] + jnp.where(m, v, zeros16))
                out.append(carry[3 * k + 4] + jnp.where(m, a, zeros16))
            return tuple(out)

        def make_body(conf_v, acc_v):
            def body(i, carry):
                v0 = conf_v[pl.ds(i * 32, 16)]
                a0 = acc_v[pl.ds(i * 32, 16)]
                v1 = conf_v[pl.ds(i * 32 + 16, 16)]
                a1 = acc_v[pl.ds(i * 32 + 16, 16)]
                return accumulate(accumulate(carry, v0, a0), v1, a1)
            return body

        init = tuple(zeros16 for _ in range(2 + 3 * (_N_BINS - 1)))
        cp[0].wait()
        cp[1].wait()
        mid = lax.fori_loop(0, half_w // 32, make_body(conf_v0, acc_v0), init)
        cp[2].wait()
        cp[3].wait()
        final = lax.fori_loop(0, half_w // 32, make_body(conf_v1, acc_v1), mid)
        for j in range(2 + 3 * (_N_BINS - 1)):
            accum_v[pl.ds(j * 16, 16)] = final[j]
        pltpu.sync_copy(accum_v, out_hbm.at[wid])

    return hist


def kernel(y, labels):
    n, _ = y.shape
    block_rows = 32768
    n_chunks = 2
    nb_total = n // block_rows
    nb = nb_total // n_chunks
    labels3 = labels.reshape(nb_total, 1, block_rows)
    bounds = jnp.linspace(0.0, 1.0, _N_BINS + 1)
    bounds_b = jnp.broadcast_to(bounds[:, None], (_N_BINS + 1, 16))
    hist = _make_hist(n // n_chunks)
    partial_list = []
    for ci in range(n_chunks):
        conf_flat, acc_flat = _stage1(y, labels3, block_rows, ci * nb, nb)
        partial_list.append(hist(conf_flat, acc_flat, bounds_b))
    partials = sum(partial_list)
    p = partials.reshape(32, 2 + 3 * (_N_BINS - 1), 16).sum(axis=(0, 2))
    tot_v, tot_a = p[0], p[1]
    exc = p[2:].reshape(_N_BINS - 1, 3)
    c_exc = jnp.concatenate([jnp.array([float(n)]), exc[:, 0],
                             jnp.array([0.0])])
    s_exc = jnp.concatenate([tot_v[None], exc[:, 1], jnp.array([0.0])])
    a_exc = jnp.concatenate([tot_a[None], exc[:, 2], jnp.array([0.0])])
    cnt = c_exc[:-1] - c_exc[1:]
    sconf = s_exc[:-1] - s_exc[1:]
    sacc = a_exc[:-1] - a_exc[1:]
    denom = jnp.maximum(cnt, 1.0)
    contrib = jnp.abs(sconf / denom - sacc / denom) * (cnt / n)
    ece = jnp.sum(jnp.where(cnt > 0, contrib, 0.0))
    return ece.reshape(1)
